# baseline (speedup 1.0000x reference)
import functools

import jax
import jax.numpy as jnp
from jax import lax
from jax.experimental import pallas as pl
from jax.experimental.pallas import tpu as pltpu
from jax.experimental.pallas import tpu_sc as plsc

NUM_CLASSES = 19
_B, _H, _W = 16, 512, 512
_CELLS = _B * (_H // 8) * (_W // 8)
_INV_N = 1.0 / (_CELLS * NUM_CLASSES)
_NW = 32
_CELL_ROWS = _B * (_H // 8)
_CRPW = _CELL_ROWS // _NW
_CPW = _CELLS // _NW
_FPW = _CPW * NUM_CLASSES


def _tc_rowor_body(t_ref, r_ref):
    t = t_ref[0]
    m = jnp.left_shift(1, t)
    a3 = m.reshape(_H // 8, 8, _W)
    r01 = a3[:, 0, :] | a3[:, 1, :]
    r23 = a3[:, 2, :] | a3[:, 3, :]
    r45 = a3[:, 4, :] | a3[:, 5, :]
    r67 = a3[:, 6, :] | a3[:, 7, :]
    r_ref[0] = (r01 | r23) | (r45 | r67)


def _sc_loss_body(r_hbm, gs_hbm, p_hbm, o_hbm, buf, obuf, pbuf, gsv, cbuf,
                  sem, psem):
    wid = lax.axis_index("s") * 2 + lax.axis_index("c")
    iota = lax.iota(jnp.int32, 16)
    rotv = [iota * 8 + ((k + (iota >> 1)) & 7) for k in range(8)]

    pltpu.async_copy(p_hbm.at[pl.ds(wid * _FPW, _FPW)], pbuf, psem)
    pltpu.sync_copy(r_hbm.at[pl.ds(wid * _CRPW, _CRPW), :], buf)
    pltpu.sync_copy(gs_hbm, gsv)

    def cellrow(rr, carry):
        rows = jnp.full((16,), 0, jnp.int32) + rr
        accs = [jnp.zeros((16,), jnp.int32) for _ in range(4)]
        for k in range(8):
            for g in range(4):
                v = plsc.load_gather(buf, [rows, rotv[k] + (g * 128)])
                accs[g] = accs[g] | v
        for g in range(4):
            obuf[pl.ds(rr * 64 + g * 16, 16)] = accs[g]
        return carry

    lax.fori_loop(0, _CRPW, cellrow, 0)

    sv = (gsv[...] - 8) & 31

    def rot(i, carry):
        m = obuf[pl.ds(i * 16, 16)]
        obuf[pl.ds(i * 16, 16)] = (
            (m << sv) | lax.shift_right_logical(m, (32 - sv) & 31))
        return carry

    lax.fori_loop(0, _CPW // 16, rot, 0)

    pltpu.make_async_copy(p_hbm.at[pl.ds(0, _FPW)], pbuf, psem).wait()

    divu = [(iota + 16 * u) // NUM_CLASSES for u in range(NUM_CLASSES)]
    cmodu = [(iota + 16 * u) % NUM_CLASSES for u in range(NUM_CLASSES)]
    ln2 = 0.6931471805599453

    def chunk(j, carry):
        t1, t2 = carry
        for u in range(NUM_CLASSES):
            p = pbuf[pl.ds(j * (16 * NUM_CLASSES) + u * 16, 16)]
            mv = plsc.load_gather(obuf, [divu[u] + j * 16])
            y = (lax.shift_right_logical(mv, cmodu[u]) & 1).astype(
                jnp.float32)
            a = jnp.abs(p)
            e = jnp.exp(-a)
            ser = e * (1.0 - e * (0.5 - e * (
                (1.0 / 3.0) - e * (0.25 - e * (0.2 - e * (
                    (1.0 / 6.0) - e * (1.0 / 7.0)))))))
            th = a * 0.5
            tq = th * th
            gsm = (ln2 - th) + tq * (0.5 + tq * (
                (-1.0 / 12.0) + tq * ((1.0 / 45.0) + tq * (-17.0 / 2520.0))))
            g = jnp.where(a <= 1.0, gsm, ser)
            sp = jnp.maximum(p, 0.0) + g
            t1 = t1 + jnp.minimum(sp, 100.0)
            t2 = t2 + y * p
        return t1, t2

    zero = jnp.zeros((16,), jnp.float32)
    t1, t2 = lax.fori_loop(0, _FPW // (16 * NUM_CLASSES), chunk, (zero, zero))
    cbuf[pl.ds(0, 16)] = t1
    cbuf[pl.ds(16, 16)] = t2
    pltpu.sync_copy(cbuf, o_hbm.at[wid])


_sc_loss = functools.partial(
    pl.kernel,
    out_type=jax.ShapeDtypeStruct((_NW, 32), jnp.float32),
    mesh=plsc.VectorSubcoreMesh(core_axis_name="c", subcore_axis_name="s"),
    scratch_types=[
        pltpu.VMEM((_CRPW, _W), jnp.int32),
        pltpu.VMEM((_CPW,), jnp.int32),
        pltpu.VMEM((_FPW,), jnp.float32),
        pltpu.VMEM((16,), jnp.int32),
        pltpu.VMEM((32,), jnp.float32),
        pltpu.SemaphoreType.DMA,
        pltpu.SemaphoreType.DMA,
    ],
    compiler_params=pltpu.CompilerParams(
        needs_layout_passes=False, use_tc_tiling_on_sc=True),
)(_sc_loss_body)


def kernel(preds, targets, grid_size):
    rowor = pl.pallas_call(
        _tc_rowor_body,
        grid=(_B,),
        in_specs=[pl.BlockSpec((1, _H, _W), lambda b: (b, 0, 0))],
        out_specs=pl.BlockSpec((1, _H // 8, _W), lambda b: (b, 0, 0)),
        out_shape=jax.ShapeDtypeStruct((_B, _H // 8, _W), jnp.int32),
    )(targets)
    gs16 = jnp.zeros((16,), jnp.int32) + jnp.asarray(grid_size, jnp.int32)
    parts = _sc_loss(rowor.reshape(_CELL_ROWS, _W), gs16, preds.reshape(-1))
    return (jnp.sum(parts[:, :16]) - jnp.sum(parts[:, 16:])) * _INV_N


# speedup vs baseline: 1.6811x; 1.6811x over previous
import functools

import jax
import jax.numpy as jnp
from jax import lax
from jax.experimental import pallas as pl
from jax.experimental.pallas import tpu as pltpu
from jax.experimental.pallas import tpu_sc as plsc

NUM_CLASSES = 19
_B, _H, _W = 16, 512, 512
_CELLS = _B * (_H // 8) * (_W // 8)
_INV_N = 1.0 / (_CELLS * NUM_CLASSES)
_NW = 32
_CELL_ROWS = _B * (_H // 8)
_CRPW = _CELL_ROWS // _NW
_CPW = _CELLS // _NW
_CPB = _CELLS // _B


def _tc_rowor_body(t_ref, r_ref):
    t = t_ref[0]
    m = jnp.left_shift(1, t)
    a3 = m.reshape(_H // 8, 8, _W)
    r01 = a3[:, 0, :] | a3[:, 1, :]
    r23 = a3[:, 2, :] | a3[:, 3, :]
    r45 = a3[:, 4, :] | a3[:, 5, :]
    r67 = a3[:, 6, :] | a3[:, 7, :]
    r_ref[0] = (r01 | r23) | (r45 | r67)


def _sc_mask_body(r_hbm, m_hbm, buf, obuf, sem):
    wid = lax.axis_index("s") * 2 + lax.axis_index("c")
    iota = lax.iota(jnp.int32, 16)
    rotv = [iota * 8 + ((k + (iota >> 1)) & 7) for k in range(8)]

    pltpu.sync_copy(r_hbm.at[pl.ds(wid * _CRPW, _CRPW), :], buf)

    def cellrow(rr, carry):
        rows = jnp.full((16,), 0, jnp.int32) + rr
        accs = [jnp.zeros((16,), jnp.int32) for _ in range(4)]
        for k in range(8):
            for g in range(4):
                v = plsc.load_gather(buf, [rows, rotv[k] + (g * 128)])
                accs[g] = accs[g] | v
        for g in range(4):
            obuf[pl.ds(rr * 64 + g * 16, 16)] = accs[g]
        return carry

    lax.fori_loop(0, _CRPW, cellrow, 0)
    pltpu.sync_copy(obuf, m_hbm.at[pl.ds(wid * _CPW, _CPW)])


_sc_masks = functools.partial(
    pl.kernel,
    out_type=jax.ShapeDtypeStruct((_CELLS,), jnp.int32),
    mesh=plsc.VectorSubcoreMesh(core_axis_name="c", subcore_axis_name="s"),
    scratch_types=[
        pltpu.VMEM((_CRPW, _W), jnp.int32),
        pltpu.VMEM((_CPW,), jnp.int32),
        pltpu.SemaphoreType.DMA,
    ],
    compiler_params=pltpu.CompilerParams(
        needs_layout_passes=False, use_tc_tiling_on_sc=True),
)(_sc_mask_body)


def _tc_combine_body(gs_ref, m_ref, p_ref, o_ref):
    b = pl.program_id(0)
    s = (gs_ref[0] - 8) & 31
    m = m_ref[0, 0].astype(jnp.uint32)
    mrot = ((m << s) | (m >> ((32 - s) & 31))).astype(jnp.int32)

    p = p_ref[...]
    sp = jnp.maximum(p, 0.0) + jnp.log1p(jnp.exp(-jnp.abs(p)))
    term1 = jnp.sum(jnp.minimum(sp, 100.0))
    cidx = lax.broadcasted_iota(jnp.int32, (_CPB, NUM_CLASSES), 1)
    y = (jnp.right_shift(mrot[:, None], cidx) & 1).astype(jnp.float32)
    term2 = -jnp.sum(y * p)

    @pl.when(b == 0)
    def _():
        o_ref[...] = jnp.zeros((1, 1), jnp.float32)

    o_ref[...] += jnp.full((1, 1), (term1 + term2) * _INV_N)


def kernel(preds, targets, grid_size):
    rowor = pl.pallas_call(
        _tc_rowor_body,
        grid=(_B,),
        in_specs=[pl.BlockSpec((1, _H, _W), lambda b: (b, 0, 0))],
        out_specs=pl.BlockSpec((1, _H // 8, _W), lambda b: (b, 0, 0)),
        out_shape=jax.ShapeDtypeStruct((_B, _H // 8, _W), jnp.int32),
    )(targets)
    masks = _sc_masks(rowor.reshape(_CELL_ROWS, _W))
    m3 = masks.reshape(_B, 1, _CPB)
    gs = jnp.asarray(grid_size, jnp.int32).reshape(1)
    out = pl.pallas_call(
        _tc_combine_body,
        grid=(_B,),
        in_specs=[
            pl.BlockSpec(memory_space=pltpu.SMEM),
            pl.BlockSpec((1, 1, _CPB), lambda b: (b, 0, 0)),
            pl.BlockSpec((_CPB, NUM_CLASSES), lambda b: (b, 0)),
        ],
        out_specs=pl.BlockSpec((1, 1), lambda b: (0, 0)),
        out_shape=jax.ShapeDtypeStruct((1, 1), jnp.float32),
    )(gs, m3, preds)
    return out[0, 0]
